# baseline (speedup 1.0000x reference)
import jax
import jax.numpy as jnp
from jax import lax
from jax.experimental import pallas as pl
from jax.experimental.pallas import tpu as pltpu
from jax.experimental.pallas import tpu_sc as plsc

P = 10
D = 128
N = 10000
E = 160000

NC = 2
NS = 16
NW = NC * NS

EA = 89600
EB = E - EA

GK = 5
SK = 5

ZB = 624
ZTAIL = N - NS * ZB

_MESH = dict(core_axis_name="c", subcore_axis_name="s")


def _make_gather(nrows, gb):
    per = nrows // NW
    gc = per // gb
    nout = gc // GK

    def body(table, idx_hbm, out, idx_v, rows_v, sem_g, sem_s):
        wid = lax.axis_index("s") * NC + lax.axis_index("c")
        pltpu.sync_copy(idx_hbm.at[wid], idx_v)
        base = wid * per

        def fire_g(grp, half):
            for b in range(GK):
                pltpu.async_copy(table.at[idx_v.at[grp * GK + b]],
                                 rows_v.at[half, b], sem_g)

        def wait_g(grp, half):
            for b in range(GK):
                pltpu.make_async_copy(table.at[idx_v.at[grp * GK + b]],
                                      rows_v.at[half, b], sem_g).wait()

        def fire_s(grp, half):
            for b in range(GK):
                j = grp * GK + b
                pltpu.async_copy(rows_v.at[half, b],
                                 out.at[pl.ds(base + j * gb, gb)], sem_s)

        def wait_s(grp, half):
            for b in range(GK):
                j = grp * GK + b
                pltpu.make_async_copy(rows_v.at[half, b],
                                      out.at[pl.ds(base + j * gb, gb)], sem_s).wait()

        fire_g(0, 0)

        def step(j0, _):
            p = lax.rem(j0, 2)
            q = 1 - p

            @pl.when(j0 > 0)
            def _drain_prev_stores():
                wait_s(j0 - 1, q)

            @pl.when(j0 + 1 < nout)
            def _fire_next_gathers():
                fire_g(j0 + 1, q)

            wait_g(j0, p)
            fire_s(j0, p)
            return 0

        lax.fori_loop(0, nout, step, 0)
        wait_s(nout - 1, (nout - 1) % 2)

    def call(table, idx3):
        return pl.kernel(
            body,
            out_type=jax.ShapeDtypeStruct((nrows, D), jnp.float32),
            mesh=plsc.VectorSubcoreMesh(**_MESH),
            scratch_types=[
                pltpu.VMEM((gc, gb), jnp.int32),
                pltpu.VMEM((2, GK, gb, D), jnp.float32),
                pltpu.SemaphoreType.DMA,
                pltpu.SemaphoreType.DMA,
            ],
        )(table, idx3)

    return call


def _make_scatter(nedges, sb):
    per = nedges // NW
    sch = per // sb
    nout = sch // SK

    def body(e2, dst_hbm, zeros_hbm, out, dst_v, e_v, sem, sem_a, agg_sh):
        c = lax.axis_index("c")
        s = lax.axis_index("s")
        wid = s * NC + c
        pltpu.sync_copy(zeros_hbm.at[pl.ds(0, ZB)], agg_sh.at[pl.ds(s * ZB, ZB)])

        @pl.when(s == 0)
        def _zero_tail():
            pltpu.sync_copy(zeros_hbm.at[pl.ds(0, ZTAIL)],
                            agg_sh.at[pl.ds(NS * ZB, ZTAIL)])

        pltpu.sync_copy(dst_hbm.at[wid], dst_v)
        plsc.subcore_barrier()
        base = wid * per

        def step(j0, _):
            gets = []
            for b in range(SK):
                j = j0 * SK + b
                gets.append(pltpu.async_copy(e2.at[pl.ds(base + j * sb, sb)],
                                             e_v.at[b], sem))
            adds = []
            for b in range(SK):
                gets[b].wait()
                adds.append(pltpu.async_copy(e_v.at[b],
                                             agg_sh.at[dst_v.at[j0 * SK + b]],
                                             sem_a, add=True))
            for b in range(SK):
                adds[b].wait()
            return 0

        lax.fori_loop(0, nout, step, 0)
        plsc.subcore_barrier()
        pltpu.sync_copy(agg_sh.at[pl.ds(s * ZB, ZB)],
                        out.at[c].at[pl.ds(s * ZB, ZB)])

        @pl.when(s == 0)
        def _read_tail():
            pltpu.sync_copy(agg_sh.at[pl.ds(NS * ZB, ZTAIL)],
                            out.at[c].at[pl.ds(NS * ZB, ZTAIL)])

    def call(e2, dst3, zeros):
        return pl.kernel(
            body,
            out_type=jax.ShapeDtypeStruct((NC, N, D), jnp.float32),
            mesh=plsc.VectorSubcoreMesh(**_MESH),
            scratch_types=[
                pltpu.VMEM((sch, sb), jnp.int32),
                pltpu.VMEM((SK, sb, D), jnp.float32),
                pltpu.SemaphoreType.DMA,
                pltpu.SemaphoreType.DMA,
                pltpu.VMEM_SHARED((N, D), jnp.float32),
            ],
        )(e2, dst3, zeros)

    return call


_gather_a = _make_gather(2 * EA, 56)
_gather_b = _make_gather(2 * EB, 88)
_scatter_a = _make_scatter(EA, 40)
_scatter_b = _make_scatter(EB, 40)

GBA, GCA = 56, (2 * EA // NW) // 56
GBB, GCB = 88, (2 * EB // NW) // 88
SBA, SCA = 40, (EA // NW) // 40
SBB, SCB = 40, (EB // NW) // 40


def _silu(v):
    return v * (1.0 / (1.0 + jnp.exp(-v)))


def _mlp_tail(h3, g, beta):
    mu = jnp.mean(h3, axis=-1, keepdims=True)
    dlt = h3 - mu
    var = jnp.mean(dlt * dlt, axis=-1, keepdims=True)
    return dlt * lax.rsqrt(var + 1e-5) * g + beta


def _edge_block(e_ref, gs_ref, gd_ref, w1e, w2, w3, b1, b2, b3, g, beta, out_ref):
    e = e_ref[...]
    pre = jnp.dot(e, w1e[...], preferred_element_type=jnp.float32)
    pre = pre + gs_ref[...] + gd_ref[...] + b1[...]
    h1 = _silu(pre)
    h2 = _silu(jnp.dot(h1, w2[...], preferred_element_type=jnp.float32) + b2[...])
    h3 = jnp.dot(h2, w3[...], preferred_element_type=jnp.float32) + b3[...]
    out_ref[...] = _mlp_tail(h3, g[...], beta[...]) + e


BE = 6400


def _tc_edge(e, e_off, g2, nblk, w1e, w2, w3, b1, b2, b3, g, beta):
    fix = lambda i: (0, 0)
    return pl.pallas_call(
        _edge_block,
        grid=(nblk,),
        in_specs=[
            pl.BlockSpec((BE, D), lambda i: (i + e_off, 0)),
            pl.BlockSpec((BE, D), lambda i: (i, 0)),
            pl.BlockSpec((BE, D), lambda i: (i + nblk, 0)),
            pl.BlockSpec((D, D), fix),
            pl.BlockSpec((D, D), fix),
            pl.BlockSpec((D, D), fix),
            pl.BlockSpec((1, D), fix),
            pl.BlockSpec((1, D), fix),
            pl.BlockSpec((1, D), fix),
            pl.BlockSpec((1, D), fix),
            pl.BlockSpec((1, D), fix),
        ],
        out_specs=pl.BlockSpec((BE, D), lambda i: (i, 0)),
        out_shape=jax.ShapeDtypeStruct((nblk * BE, D), jnp.float32),
    )(e, g2, g2, w1e, w2, w3, b1, b2, b3, g, beta)


def _node_block(aggA_ref, aggB_ref, x_ref, w1a, w1x, w2, w3, b1, b2, b3,
                g, beta, wys, wyd, xo_ref, y_ref):
    x = x_ref[...]
    agg = aggA_ref[0] + aggA_ref[1] + aggB_ref[0] + aggB_ref[1]
    pre = (jnp.dot(agg, w1a[...], preferred_element_type=jnp.float32)
           + jnp.dot(x, w1x[...], preferred_element_type=jnp.float32) + b1[...])
    h1 = _silu(pre)
    h2 = _silu(jnp.dot(h1, w2[...], preferred_element_type=jnp.float32) + b2[...])
    h3 = jnp.dot(h2, w3[...], preferred_element_type=jnp.float32) + b3[...]
    xo = _mlp_tail(h3, g[...], beta[...]) + x
    xo_ref[...] = xo
    y_ref[0] = jnp.dot(xo, wys[...], preferred_element_type=jnp.float32)
    y_ref[1] = jnp.dot(xo, wyd[...], preferred_element_type=jnp.float32)


BN = 2000


def _tc_node(aggA, aggB, x, w1a, w1x, w2, w3, b1, b2, b3, g, beta, wys, wyd):
    row = lambda i: (i, 0)
    fix = lambda i: (0, 0)
    pair = lambda i: (0, i, 0)
    return pl.pallas_call(
        _node_block,
        grid=(N // BN,),
        in_specs=[
            pl.BlockSpec((2, BN, D), pair),
            pl.BlockSpec((2, BN, D), pair),
            pl.BlockSpec((BN, D), row),
            pl.BlockSpec((D, D), fix),
            pl.BlockSpec((D, D), fix),
            pl.BlockSpec((D, D), fix),
            pl.BlockSpec((D, D), fix),
            pl.BlockSpec((1, D), fix),
            pl.BlockSpec((1, D), fix),
            pl.BlockSpec((1, D), fix),
            pl.BlockSpec((1, D), fix),
            pl.BlockSpec((1, D), fix),
            pl.BlockSpec((D, D), fix),
            pl.BlockSpec((D, D), fix),
        ],
        out_specs=[pl.BlockSpec((BN, D), row), pl.BlockSpec((2, BN, D), pair)],
        out_shape=[jax.ShapeDtypeStruct((N, D), jnp.float32),
                   jax.ShapeDtypeStruct((2, N, D), jnp.float32)],
    )(aggA, aggB, x, w1a, w1x, w2, w3, b1, b2, b3, g, beta, wys, wyd)


def _init_block(x_ref, wys, wyd, y_ref):
    x = x_ref[...]
    y_ref[0] = jnp.dot(x, wys[...], preferred_element_type=jnp.float32)
    y_ref[1] = jnp.dot(x, wyd[...], preferred_element_type=jnp.float32)


def _tc_init(x, wys, wyd):
    return pl.pallas_call(
        _init_block,
        grid=(N // BN,),
        in_specs=[
            pl.BlockSpec((BN, D), lambda i: (i, 0)),
            pl.BlockSpec((D, D), lambda i: (0, 0)),
            pl.BlockSpec((D, D), lambda i: (0, 0)),
        ],
        out_specs=pl.BlockSpec((2, BN, D), lambda i: (0, i, 0)),
        out_shape=jax.ShapeDtypeStruct((2, N, D), jnp.float32),
    )(x, wys, wyd)


def kernel(node_features, edge_features, edge_index, eW1, eb1, eW2, eb2,
           eW3, eb3, eg, ebeta, nW1, nb1, nW2, nb2, nW3, nb3, ng, nbeta):
    src = edge_index[0]
    dst = edge_index[1]
    idxA = jnp.concatenate([src[:EA], dst[:EA] + N]).reshape(NW, GCA, GBA)
    idxB = jnp.concatenate([src[EA:], dst[EA:] + N]).reshape(NW, GCB, GBB)
    dstA3 = dst[:EA].reshape(NW, SCA, SBA)
    dstB3 = dst[EA:].reshape(NW, SCB, SBB)
    zeros = jnp.zeros((ZB, D), jnp.float32)

    W1e = eW1[:, :D]
    W1s = eW1[:, D:2 * D]
    W1d = eW1[:, 2 * D:]
    nW1a = nW1[:, :D]
    nW1x = nW1[:, D:]
    r1 = lambda v: v.reshape(1, D)

    x = node_features
    eA = edge_features
    eB = edge_features
    eB_off = EA // BE
    y = _tc_init(x, W1s[0], W1d[0])
    for i in range(P):
        table = y.reshape(2 * N, D)
        ew = (W1e[i], eW2[i], eW3[i],
              r1(eb1[i]), r1(eb2[i]), r1(eb3[i]), r1(eg[i]), r1(ebeta[i]))
        gA = _gather_a(table, idxA)
        gB = _gather_b(table, idxB)
        eA = _tc_edge(eA, 0, gA, EA // BE, *ew)
        aggA = _scatter_a(eA, dstA3, zeros)
        eB = _tc_edge(eB, eB_off, gB, EB // BE, *ew)
        eB_off = 0
        aggB = _scatter_b(eB, dstB3, zeros)
        j = min(i + 1, P - 1)
        x, y = _tc_node(aggA, aggB, x, nW1a[i], nW1x[i], nW2[i], nW3[i],
                        r1(nb1[i]), r1(nb2[i]), r1(nb3[i]), r1(ng[i]), r1(nbeta[i]),
                        W1s[j], W1d[j])
    return x


# speedup vs baseline: 1.0098x; 1.0098x over previous
import jax
import jax.numpy as jnp
from jax import lax
from jax.experimental import pallas as pl
from jax.experimental.pallas import tpu as pltpu
from jax.experimental.pallas import tpu_sc as plsc

P = 10
D = 128
N = 10000
E = 160000

NC = 2
NS = 16
NW = NC * NS

EA = 102400
EB = E - EA

GK = 5
SK = 5

ZB = 624
ZTAIL = N - NS * ZB

_MESH = dict(core_axis_name="c", subcore_axis_name="s")


def _make_gather(nrows, gb):
    per = nrows // NW
    gc = per // gb
    nout = gc // GK

    def body(table, idx_hbm, out, idx_v, rows_v, sem_g, sem_s):
        wid = lax.axis_index("s") * NC + lax.axis_index("c")
        pltpu.sync_copy(idx_hbm.at[wid], idx_v)
        base = wid * per

        def fire_g(grp, half):
            for b in range(GK):
                pltpu.async_copy(table.at[idx_v.at[grp * GK + b]],
                                 rows_v.at[half, b], sem_g)

        def wait_g(grp, half):
            for b in range(GK):
                pltpu.make_async_copy(table.at[idx_v.at[grp * GK + b]],
                                      rows_v.at[half, b], sem_g).wait()

        def fire_s(grp, half):
            for b in range(GK):
                j = grp * GK + b
                pltpu.async_copy(rows_v.at[half, b],
                                 out.at[pl.ds(base + j * gb, gb)], sem_s)

        def wait_s(grp, half):
            for b in range(GK):
                j = grp * GK + b
                pltpu.make_async_copy(rows_v.at[half, b],
                                      out.at[pl.ds(base + j * gb, gb)], sem_s).wait()

        fire_g(0, 0)

        def step(j0, _):
            p = lax.rem(j0, 2)
            q = 1 - p

            @pl.when(j0 > 0)
            def _drain_prev_stores():
                wait_s(j0 - 1, q)

            @pl.when(j0 + 1 < nout)
            def _fire_next_gathers():
                fire_g(j0 + 1, q)

            wait_g(j0, p)
            fire_s(j0, p)
            return 0

        lax.fori_loop(0, nout, step, 0)
        wait_s(nout - 1, (nout - 1) % 2)

    def call(table, idx3):
        return pl.kernel(
            body,
            out_type=jax.ShapeDtypeStruct((nrows, D), jnp.float32),
            mesh=plsc.VectorSubcoreMesh(**_MESH),
            scratch_types=[
                pltpu.VMEM((gc, gb), jnp.int32),
                pltpu.VMEM((2, GK, gb, D), jnp.float32),
                pltpu.SemaphoreType.DMA,
                pltpu.SemaphoreType.DMA,
            ],
        )(table, idx3)

    return call


def _make_scatter(nedges, sb):
    per = nedges // NW
    sch = per // sb
    nout = sch // SK

    def body(e2, dst_hbm, zeros_hbm, out, dst_v, e_v, sem, sem_a, agg_sh):
        c = lax.axis_index("c")
        s = lax.axis_index("s")
        wid = s * NC + c
        pltpu.sync_copy(zeros_hbm.at[pl.ds(0, ZB)], agg_sh.at[pl.ds(s * ZB, ZB)])

        @pl.when(s == 0)
        def _zero_tail():
            pltpu.sync_copy(zeros_hbm.at[pl.ds(0, ZTAIL)],
                            agg_sh.at[pl.ds(NS * ZB, ZTAIL)])

        pltpu.sync_copy(dst_hbm.at[wid], dst_v)
        plsc.subcore_barrier()
        base = wid * per

        def step(j0, _):
            gets = []
            for b in range(SK):
                j = j0 * SK + b
                gets.append(pltpu.async_copy(e2.at[pl.ds(base + j * sb, sb)],
                                             e_v.at[b], sem))
            adds = []
            for b in range(SK):
                gets[b].wait()
                adds.append(pltpu.async_copy(e_v.at[b],
                                             agg_sh.at[dst_v.at[j0 * SK + b]],
                                             sem_a, add=True))
            for b in range(SK):
                adds[b].wait()
            return 0

        lax.fori_loop(0, nout, step, 0)
        plsc.subcore_barrier()
        pltpu.sync_copy(agg_sh.at[pl.ds(s * ZB, ZB)],
                        out.at[c].at[pl.ds(s * ZB, ZB)])

        @pl.when(s == 0)
        def _read_tail():
            pltpu.sync_copy(agg_sh.at[pl.ds(NS * ZB, ZTAIL)],
                            out.at[c].at[pl.ds(NS * ZB, ZTAIL)])

    def call(e2, dst3, zeros):
        return pl.kernel(
            body,
            out_type=jax.ShapeDtypeStruct((NC, N, D), jnp.float32),
            mesh=plsc.VectorSubcoreMesh(**_MESH),
            scratch_types=[
                pltpu.VMEM((sch, sb), jnp.int32),
                pltpu.VMEM((SK, sb, D), jnp.float32),
                pltpu.SemaphoreType.DMA,
                pltpu.SemaphoreType.DMA,
                pltpu.VMEM_SHARED((N, D), jnp.float32),
            ],
        )(e2, dst3, zeros)

    return call


_gather_a = _make_gather(2 * EA, 80)
_gather_b = _make_gather(2 * EB, 72)
_scatter_a = _make_scatter(EA, 64)
_scatter_b = _make_scatter(EB, 40)

GBA, GCA = 80, (2 * EA // NW) // 80
GBB, GCB = 72, (2 * EB // NW) // 72
SBA, SCA = 64, (EA // NW) // 64
SBB, SCB = 40, (EB // NW) // 40


def _silu(v):
    return v * (1.0 / (1.0 + jnp.exp(-v)))


def _mlp_tail(h3, g, beta):
    mu = jnp.mean(h3, axis=-1, keepdims=True)
    dlt = h3 - mu
    var = jnp.mean(dlt * dlt, axis=-1, keepdims=True)
    return dlt * lax.rsqrt(var + 1e-5) * g + beta


def _edge_block(e_ref, gs_ref, gd_ref, w1e, w2, w3, b1, b2, b3, g, beta, out_ref):
    e = e_ref[...]
    pre = jnp.dot(e, w1e[...], preferred_element_type=jnp.float32)
    pre = pre + gs_ref[...] + gd_ref[...] + b1[...]
    h1 = _silu(pre)
    h2 = _silu(jnp.dot(h1, w2[...], preferred_element_type=jnp.float32) + b2[...])
    h3 = jnp.dot(h2, w3[...], preferred_element_type=jnp.float32) + b3[...]
    out_ref[...] = _mlp_tail(h3, g[...], beta[...]) + e


BE = 6400


def _tc_edge(e, e_off, g2, nblk, w1e, w2, w3, b1, b2, b3, g, beta):
    fix = lambda i: (0, 0)
    return pl.pallas_call(
        _edge_block,
        grid=(nblk,),
        in_specs=[
            pl.BlockSpec((BE, D), lambda i: (i + e_off, 0)),
            pl.BlockSpec((BE, D), lambda i: (i, 0)),
            pl.BlockSpec((BE, D), lambda i: (i + nblk, 0)),
            pl.BlockSpec((D, D), fix),
            pl.BlockSpec((D, D), fix),
            pl.BlockSpec((D, D), fix),
            pl.BlockSpec((1, D), fix),
            pl.BlockSpec((1, D), fix),
            pl.BlockSpec((1, D), fix),
            pl.BlockSpec((1, D), fix),
            pl.BlockSpec((1, D), fix),
        ],
        out_specs=pl.BlockSpec((BE, D), lambda i: (i, 0)),
        out_shape=jax.ShapeDtypeStruct((nblk * BE, D), jnp.float32),
    )(e, g2, g2, w1e, w2, w3, b1, b2, b3, g, beta)


def _node_block(aggA_ref, aggB_ref, x_ref, w1a, w1x, w2, w3, b1, b2, b3,
                g, beta, wys, wyd, xo_ref, y_ref):
    x = x_ref[...]
    agg = aggA_ref[0] + aggA_ref[1] + aggB_ref[0] + aggB_ref[1]
    pre = (jnp.dot(agg, w1a[...], preferred_element_type=jnp.float32)
           + jnp.dot(x, w1x[...], preferred_element_type=jnp.float32) + b1[...])
    h1 = _silu(pre)
    h2 = _silu(jnp.dot(h1, w2[...], preferred_element_type=jnp.float32) + b2[...])
    h3 = jnp.dot(h2, w3[...], preferred_element_type=jnp.float32) + b3[...]
    xo = _mlp_tail(h3, g[...], beta[...]) + x
    xo_ref[...] = xo
    y_ref[0] = jnp.dot(xo, wys[...], preferred_element_type=jnp.float32)
    y_ref[1] = jnp.dot(xo, wyd[...], preferred_element_type=jnp.float32)


BN = 2000


def _tc_node(aggA, aggB, x, w1a, w1x, w2, w3, b1, b2, b3, g, beta, wys, wyd):
    row = lambda i: (i, 0)
    fix = lambda i: (0, 0)
    pair = lambda i: (0, i, 0)
    return pl.pallas_call(
        _node_block,
        grid=(N // BN,),
        in_specs=[
            pl.BlockSpec((2, BN, D), pair),
            pl.BlockSpec((2, BN, D), pair),
            pl.BlockSpec((BN, D), row),
            pl.BlockSpec((D, D), fix),
            pl.BlockSpec((D, D), fix),
            pl.BlockSpec((D, D), fix),
            pl.BlockSpec((D, D), fix),
            pl.BlockSpec((1, D), fix),
            pl.BlockSpec((1, D), fix),
            pl.BlockSpec((1, D), fix),
            pl.BlockSpec((1, D), fix),
            pl.BlockSpec((1, D), fix),
            pl.BlockSpec((D, D), fix),
            pl.BlockSpec((D, D), fix),
        ],
        out_specs=[pl.BlockSpec((BN, D), row), pl.BlockSpec((2, BN, D), pair)],
        out_shape=[jax.ShapeDtypeStruct((N, D), jnp.float32),
                   jax.ShapeDtypeStruct((2, N, D), jnp.float32)],
    )(aggA, aggB, x, w1a, w1x, w2, w3, b1, b2, b3, g, beta, wys, wyd)


def _init_block(x_ref, wys, wyd, y_ref):
    x = x_ref[...]
    y_ref[0] = jnp.dot(x, wys[...], preferred_element_type=jnp.float32)
    y_ref[1] = jnp.dot(x, wyd[...], preferred_element_type=jnp.float32)


def _tc_init(x, wys, wyd):
    return pl.pallas_call(
        _init_block,
        grid=(N // BN,),
        in_specs=[
            pl.BlockSpec((BN, D), lambda i: (i, 0)),
            pl.BlockSpec((D, D), lambda i: (0, 0)),
            pl.BlockSpec((D, D), lambda i: (0, 0)),
        ],
        out_specs=pl.BlockSpec((2, BN, D), lambda i: (0, i, 0)),
        out_shape=jax.ShapeDtypeStruct((2, N, D), jnp.float32),
    )(x, wys, wyd)


def kernel(node_features, edge_features, edge_index, eW1, eb1, eW2, eb2,
           eW3, eb3, eg, ebeta, nW1, nb1, nW2, nb2, nW3, nb3, ng, nbeta):
    src = edge_index[0]
    dst = edge_index[1]
    idxA = jnp.concatenate([src[:EA], dst[:EA] + N]).reshape(NW, GCA, GBA)
    idxB = jnp.concatenate([src[EA:], dst[EA:] + N]).reshape(NW, GCB, GBB)
    dstA3 = dst[:EA].reshape(NW, SCA, SBA)
    dstB3 = dst[EA:].reshape(NW, SCB, SBB)
    zeros = jnp.zeros((ZB, D), jnp.float32)

    W1e = eW1[:, :D]
    W1s = eW1[:, D:2 * D]
    W1d = eW1[:, 2 * D:]
    nW1a = nW1[:, :D]
    nW1x = nW1[:, D:]
    r1 = lambda v: v.reshape(1, D)

    x = node_features
    eA = edge_features
    eB = edge_features
    eB_off = EA // BE
    y = _tc_init(x, W1s[0], W1d[0])
    for i in range(P):
        table = y.reshape(2 * N, D)
        ew = (W1e[i], eW2[i], eW3[i],
              r1(eb1[i]), r1(eb2[i]), r1(eb3[i]), r1(eg[i]), r1(ebeta[i]))
        gA = _gather_a(table, idxA)
        gB = _gather_b(table, idxB)
        eA = _tc_edge(eA, 0, gA, EA // BE, *ew)
        aggA = _scatter_a(eA, dstA3, zeros)
        eB = _tc_edge(eB, eB_off, gB, EB // BE, *ew)
        eB_off = 0
        aggB = _scatter_b(eB, dstB3, zeros)
        j = min(i + 1, P - 1)
        x, y = _tc_node(aggA, aggB, x, nW1a[i], nW1x[i], nW2[i], nW3[i],
                        r1(nb1[i]), r1(nb2[i]), r1(nb3[i]), r1(ng[i]), r1(nbeta[i]),
                        W1s[j], W1d[j])
    return x
